# baseline (speedup 1.0000x reference)
import functools

import jax
import jax.numpy as jnp
from jax import lax
from jax.experimental import pallas as pl
from jax.experimental.pallas import tpu as pltpu
from jax.experimental.pallas import tpu_sc as plsc

W_PULL = 1.0
W_PUSH = 1.0
W_NORM = 0.001
W_EDGE = 10.0
D_V = 0.5
D_D = 1.5

B = 2
E = 16
Z = 64
PLANE = 64 * 64
HPLANE = PLANE // 2
NC = 2
NS = 16
NW = NC * NS
ZPW = Z // NS
NSEG = 2 * ZPW
LPAD = 64
PART = 18 * 16
CEN = 9 * 16
MISC = 2 * 16


def _nsqrt(x):
    i = plsc.bitcast(x, jnp.int32)
    i = jnp.int32(0x5F3759DF) - lax.shift_right_logical(i, 1)
    z = plsc.bitcast(i, jnp.float32)
    z = z * (1.5 - 0.5 * x * z * z)
    z = z * (1.5 - 0.5 * x * z * z)
    z = z * (1.5 - 0.5 * x * z * z)
    return x * z


def _mesh():
    return plsc.VectorSubcoreMesh(
        core_axis_name="c", subcore_axis_name="s", num_cores=NC, num_subcores=NS)


@functools.partial(
    pl.kernel,
    out_type=[
        jax.ShapeDtypeStruct((B * 16,), jnp.float32),
        jax.ShapeDtypeStruct((B * Z * PLANE,), jnp.float32),
    ],
    mesh=_mesh(),
    compiler_params=pltpu.CompilerParams(needs_layout_passes=False),
    scratch_types=[
        pltpu.VMEM((PLANE + 2 * LPAD,), jnp.int32),
        pltpu.VMEM((PLANE + 2 * LPAD,), jnp.int32),
        pltpu.VMEM((PLANE + 2 * LPAD,), jnp.int32),
        pltpu.VMEM((PLANE,), jnp.int32),
        pltpu.VMEM((PLANE,), jnp.int32),
        pltpu.VMEM((PLANE,), jnp.int32),
        pltpu.VMEM((PLANE,), jnp.int32),
        pltpu.VMEM((PLANE,), jnp.int32),
        pltpu.VMEM((PLANE,), jnp.int32),
        pltpu.VMEM((PLANE,), jnp.int32),
        pltpu.VMEM((PLANE,), jnp.int32),
        pltpu.VMEM((PLANE,), jnp.float32),
        pltpu.VMEM((E * PLANE,), jnp.float32),
        pltpu.VMEM((16 * PART,), jnp.float32),
        pltpu.VMEM((PART,), jnp.float32),
        pltpu.VMEM((E * 16,), jnp.float32),
        pltpu.VMEM((16 * E * 16,), jnp.float32),
        pltpu.VMEM((CEN,), jnp.float32),
        pltpu.VMEM_SHARED((NS * PART,), jnp.float32),
        pltpu.SemaphoreType.DMA,
        pltpu.SemaphoreType.DMA,
    ],
)
def _kern_ab(embed, label, loss_out, w_out,
             lab0, lab1, lab2, mn0, mn1, mn2, mx0, mx1, mx2, mnx, mxx,
             wbuf, emb, rep, part_v, centers_v, crep, cent_t,
             shared, sem0, sem1):
    labs = [lab0, lab1, lab2]
    mns = [mn0, mn1, mn2]
    mxs = [mx0, mx1, mx2]
    sems = [sem0, sem1]
    c = lax.axis_index("c")
    s = lax.axis_index("s")
    wid = c * NS + s
    b = c
    z0 = s * ZPW
    io = lax.iota(jnp.int32, 16)
    zf = jnp.zeros((16,), jnp.float32)
    onef = jnp.ones((16,), jnp.float32)

    def zrow(r, _):
        rep[pl.ds(r * 16, 16)] = zf
        return 0
    lax.fori_loop(0, 16 * PART // 16, zrow, 0)

    def fire_full(k):
        z = z0 + k
        return [pltpu.async_copy(
                    embed.at[pl.ds(((b * E + e) * Z + z) * PLANE, PLANE)],
                    emb.at[pl.ds(e * PLANE, PLANE)], sems[0])
                for e in range(E)]

    def fire(t):
        k, h = t // 2, t % 2
        z = z0 + k
        eb = (t % 2) * (E * HPLANE)
        return [pltpu.async_copy(
                    embed.at[pl.ds(((b * E + e) * Z + z) * PLANE + h * HPLANE,
                                   HPLANE)],
                    emb.at[pl.ds(eb + e * HPLANE, HPLANE)], sems[t % 2])
                for e in range(E)]

    cps = fire_full(0)

    def load_lab(z, slot):
        zc = jnp.clip(z, 0, Z - 1)
        pltpu.sync_copy(label.at[pl.ds((b * Z + zc) * PLANE, PLANE)],
                        labs[slot].at[pl.ds(LPAD, PLANE)])

    def xy_pass(slot):
        lab = labs[slot]
        mnr = mns[slot]
        mxr = mxs[slot]

        @plsc.parallel_loop(0, 64, unroll=1)
        def xrow(r):
            base = LPAD + r * 64
            for p in range(4):
                o = base + p * 16
                cv = lab[pl.ds(o, 16)]
                lv = lab[pl.ds(o - 1, 16)]
                rv = lab[pl.ds(o + 1, 16)]
                if p == 0:
                    lv = jnp.where(io == 0, cv, lv)
                if p == 3:
                    rv = jnp.where(io == 15, cv, rv)
                oo = r * 64 + p * 16
                mnx[pl.ds(oo, 16)] = jnp.minimum(jnp.minimum(lv, cv), rv)
                mxx[pl.ds(oo, 16)] = jnp.maximum(jnp.maximum(lv, cv), rv)

        @plsc.parallel_loop(0, 64, unroll=2)
        def yrow(r):
            rm = jnp.maximum(r - 1, 0) * 64
            rc = r * 64
            rp = jnp.minimum(r + 1, 63) * 64
            for p in range(4):
                q = p * 16
                mnr[pl.ds(rc + q, 16)] = jnp.minimum(
                    jnp.minimum(mnx[pl.ds(rm + q, 16)], mnx[pl.ds(rc + q, 16)]),
                    mnx[pl.ds(rp + q, 16)])
                mxr[pl.ds(rc + q, 16)] = jnp.maximum(
                    jnp.maximum(mxx[pl.ds(rm + q, 16)], mxx[pl.ds(rc + q, 16)]),
                    mxx[pl.ds(rp + q, 16)])

    load_lab(z0 - 1, 0)
    xy_pass(0)
    load_lab(z0, 1)
    xy_pass(1)

    for k in range(ZPW):
        z = z0 + k
        sl_cur = (k + 1) % 3
        sl_next = (k + 2) % 3

        load_lab(z + 1, sl_next)
        xy_pass(sl_next)

        mnp_, mnc_, mnn_ = mns[k % 3], mns[sl_cur], mns[sl_next]
        mxp_, mxc_, mxn_ = mxs[k % 3], mxs[sl_cur], mxs[sl_next]
        labc_ = labs[sl_cur]

        for cp in cps:
            cp.wait()

        _mn = (mnp_, mnc_, mnn_)
        _mx = (mxp_, mxc_, mxn_)
        _lab = labc_

        @plsc.parallel_loop(0, PLANE // 16, unroll=4)
        def achunk(i):
            po = i * 16
            mnv = jnp.minimum(
                jnp.minimum(_mn[0][pl.ds(po, 16)],
                            _mn[1][pl.ds(po, 16)]),
                _mn[2][pl.ds(po, 16)])
            mxv = jnp.maximum(
                jnp.maximum(_mx[0][pl.ds(po, 16)],
                            _mx[1][pl.ds(po, 16)]),
                _mx[2][pl.ds(po, 16)])
            wv = jnp.where(mxv != mnv, W_EDGE, 1.0)
            wbuf[pl.ds(po, 16)] = wv
            labv = _lab[pl.ds(LPAD + po, 16)]
            base = labv * 16 + io
            plsc.addupdate_scatter(rep, [base + 16 * 256], wv)
            plsc.addupdate_scatter(rep, [base + 17 * 256], onef)
            for e in range(E):
                v = emb[pl.ds(e * PLANE + po, 16)]
                plsc.addupdate_scatter(rep, [base + e * 256], wv * v)
        if k + 1 < ZPW:
            cps = fire_full(k + 1)

        pltpu.sync_copy(wbuf, w_out.at[pl.ds((b * Z + z) * PLANE, PLANE)])

    cps = fire(0)

    def fold(g, _):
        acc = zf
        for l in range(16):
            acc = acc + plsc.load_gather(rep, [g * 256 + io * 16 + l])
        part_v[pl.ds(g * 16, 16)] = acc
        return 0
    lax.fori_loop(0, 18, fold, 0)

    pltpu.sync_copy(part_v, shared.at[pl.ds(s * PART, PART)])
    plsc.subcore_barrier()
    pltpu.sync_copy(shared, rep.at[pl.ds(0, NS * PART)])

    def redrow(g):
        def red(i, acc, _g=g):
            return acc + rep[pl.ds(i * PART + _g * 16, 16)]
        return lax.fori_loop(0, NS, red, zf)

    wsum_vec = redrow(16) + 1e-8
    cnt_vec = redrow(17)
    for e in range(E):
        centers_v[pl.ds(e * 16, 16)] = redrow(e) / wsum_vec

    def crow(e, _):
        row = centers_v[pl.ds(e * 16, 16)]
        for l in range(16):
            plsc.store_scatter(crep, [e * 256 + io * 16 + l], row)
        return 0
    lax.fori_loop(0, E, crow, 0)

    @pl.when(s == 0)
    def _():
        cent_t[pl.ds(0, 16)] = zf
        for uid in range(1, 9):
            cent_t[pl.ds(uid * 16, 16)] = plsc.load_gather(
                centers_v, [io * 16 + uid])

    for r in range(16):
        rep[pl.ds(r * 16, 16)] = zf

    for t in range(NSEG):
        k, h = t // 2, t % 2
        z = z0 + k
        if h == 0:
            pltpu.sync_copy(label.at[pl.ds((b * Z + z) * PLANE, PLANE)],
                            labs[0].at[pl.ds(0, PLANE)])
            pltpu.sync_copy(w_out.at[pl.ds((b * Z + z) * PLANE, PLANE)],
                            wbuf)
        for cp in cps:
            cp.wait()
        if t + 1 < NSEG:
            cps = fire(t + 1)
        eb = (t % 2) * (E * HPLANE)
        hb = h * HPLANE

        _hb, _eb = hb, eb

        @plsc.parallel_loop(0, HPLANE // 16, unroll=2)
        def pchunk(i):
            o = i * 16
            po = _hb + o
            labv = labs[0][pl.ds(po, 16)]
            wv = wbuf[pl.ds(po, 16)]
            base = labv * 16 + io
            sq = []
            for e in range(E):
                v = emb[pl.ds(_eb + e * HPLANE + o, 16)]
                cs = plsc.load_gather(crep, [base + e * 256])
                d = v - cs
                sq.append(d * d)
            while len(sq) > 1:
                sq = [a + bq for a, bq in zip(sq[::2], sq[1::2])]
            dist = _nsqrt(sq[0])
            t_ = jnp.maximum(dist - D_V, 0.0)
            ww = jnp.where(labv == 0, 0.0, wv)
            plsc.addupdate_scatter(rep, [base], t_ * t_ * ww)

    acc = zf
    for l in range(16):
        acc = acc + plsc.load_gather(rep, [io * 16 + l])
    part_v[pl.ds(0, 16)] = acc
    pltpu.sync_copy(part_v.at[pl.ds(0, 16)], shared.at[pl.ds(s * PART, 16)])
    plsc.subcore_barrier()

    @pl.when(s == 0)
    def _():
        pltpu.sync_copy(shared, rep.at[pl.ds(0, NS * PART)])
        def redp(i, a):
            return a + rep[pl.ds(i * PART, 16)]
        pull_vec = lax.fori_loop(0, NS, redp, zf)
        uidmask = (io >= 1) & (io <= 8)
        q = jnp.where(uidmask, pull_vec / cnt_vec, 0.0)
        loss = jnp.sum(q) * W_PULL
        push = jnp.float32(0.0)
        for i in range(1, 8):
            def pj(j, a, _i=i):
                d = (cent_t[pl.ds(_i * 16, 16)]
                     - cent_t[pl.ds(j * 16, 16)])
                dd = jnp.max(_nsqrt(jnp.full((16,), jnp.sum(d * d))))
                pp = jnp.maximum(2.0 * D_D - dd, 0.0)
                return a + pp * pp
            push = lax.fori_loop(i + 1, 9, pj, push)
        loss = loss + push * (W_PUSH / 28.0)
        norm = jnp.float32(0.0)
        for i in range(1, 9):
            cv = cent_t[pl.ds(i * 16, 16)]
            norm = norm + jnp.max(_nsqrt(jnp.full((16,), jnp.sum(cv * cv))))
        loss = loss + norm * (W_NORM * 0.125)
        part_v[pl.ds(0, 16)] = jnp.full((16,), loss)
        pltpu.sync_copy(part_v.at[pl.ds(0, 16)],
                        loss_out.at[pl.ds(b * 16, 16)])


def kernel(embed, label):
    embed_f = embed.reshape(-1)
    label_f = label.reshape(-1)
    loss, _w = _kern_ab(embed_f, label_f)
    return (loss[0] + loss[16]) * jnp.float32(1.0 / B)


# speedup vs baseline: 1.0646x; 1.0646x over previous
import functools

import jax
import jax.numpy as jnp
from jax import lax
from jax.experimental import pallas as pl
from jax.experimental.pallas import tpu as pltpu
from jax.experimental.pallas import tpu_sc as plsc

W_PULL = 1.0
W_PUSH = 1.0
W_NORM = 0.001
W_EDGE = 10.0
D_V = 0.5
D_D = 1.5

B = 2
E = 16
Z = 64
PLANE = 64 * 64
HPLANE = PLANE // 2
NC = 2
NS = 16
NW = NC * NS
ZPW = Z // NS
NSEG = 2 * ZPW
LPAD = 64
PART = 18 * 16
CEN = 9 * 16
MISC = 2 * 16


def _nsqrt(x):
    i = plsc.bitcast(x, jnp.int32)
    i = jnp.int32(0x5F3759DF) - lax.shift_right_logical(i, 1)
    z = plsc.bitcast(i, jnp.float32)
    z = z * (1.5 - 0.5 * x * z * z)
    z = z * (1.5 - 0.5 * x * z * z)
    z = z * (1.5 - 0.5 * x * z * z)
    return x * z


def _mesh():
    return plsc.VectorSubcoreMesh(
        core_axis_name="c", subcore_axis_name="s", num_cores=NC, num_subcores=NS)


@functools.partial(
    pl.kernel,
    out_type=[
        jax.ShapeDtypeStruct((B * 16,), jnp.float32),
        jax.ShapeDtypeStruct((B * Z * PLANE,), jnp.float32),
    ],
    mesh=_mesh(),
    compiler_params=pltpu.CompilerParams(needs_layout_passes=False),
    scratch_types=[
        pltpu.VMEM((PLANE + 2 * LPAD,), jnp.int32),
        pltpu.VMEM((PLANE + 2 * LPAD,), jnp.int32),
        pltpu.VMEM((PLANE + 2 * LPAD,), jnp.int32),
        pltpu.VMEM((PLANE,), jnp.int32),
        pltpu.VMEM((PLANE,), jnp.int32),
        pltpu.VMEM((PLANE,), jnp.int32),
        pltpu.VMEM((PLANE,), jnp.int32),
        pltpu.VMEM((PLANE,), jnp.int32),
        pltpu.VMEM((PLANE,), jnp.int32),
        pltpu.VMEM((PLANE,), jnp.int32),
        pltpu.VMEM((PLANE,), jnp.int32),
        pltpu.VMEM((PLANE,), jnp.float32),
        pltpu.VMEM((E * PLANE,), jnp.float32),
        pltpu.VMEM((16 * PART,), jnp.float32),
        pltpu.VMEM((PART,), jnp.float32),
        pltpu.VMEM((E * 16,), jnp.float32),
        pltpu.VMEM((16 * E * 16,), jnp.float32),
        pltpu.VMEM((CEN,), jnp.float32),
        pltpu.VMEM_SHARED((NS * PART,), jnp.float32),
        pltpu.SemaphoreType.DMA,
        pltpu.SemaphoreType.DMA,
    ],
)
def _kern_ab(embed, label, loss_out, w_out,
             lab0, lab1, lab2, mn0, mn1, mn2, mx0, mx1, mx2, mnx, mxx,
             wbuf, emb, rep, part_v, centers_v, crep, cent_t,
             shared, sem0, sem1):
    labs = [lab0, lab1, lab2]
    mns = [mn0, mn1, mn2]
    mxs = [mx0, mx1, mx2]
    sems = [sem0, sem1]
    c = lax.axis_index("c")
    s = lax.axis_index("s")
    wid = c * NS + s
    b = c
    z0 = s * ZPW
    io = lax.iota(jnp.int32, 16)
    zf = jnp.zeros((16,), jnp.float32)
    onef = jnp.ones((16,), jnp.float32)

    def zrow(r, _):
        rep[pl.ds(r * 16, 16)] = zf
        return 0
    lax.fori_loop(0, 16 * PART // 16, zrow, 0)

    def fire_full(k):
        z = z0 + k
        return [pltpu.async_copy(
                    embed.at[pl.ds(((b * E + e) * Z + z) * PLANE, PLANE)],
                    emb.at[pl.ds(e * PLANE, PLANE)], sems[0])
                for e in range(E)]

    def fire(t):
        k, h = t // 2, t % 2
        z = z0 + k
        eb = (t % 2) * (E * HPLANE)
        return [pltpu.async_copy(
                    embed.at[pl.ds(((b * E + e) * Z + z) * PLANE + h * HPLANE,
                                   HPLANE)],
                    emb.at[pl.ds(eb + e * HPLANE, HPLANE)], sems[t % 2])
                for e in range(E)]

    cps = fire_full(0)

    def load_lab(z, slot):
        zc = jnp.clip(z, 0, Z - 1)
        pltpu.sync_copy(label.at[pl.ds((b * Z + zc) * PLANE, PLANE)],
                        labs[slot].at[pl.ds(LPAD, PLANE)])

    def xy_pass(slot):
        lab = labs[slot]
        mnr = mns[slot]
        mxr = mxs[slot]

        @plsc.parallel_loop(0, 64, unroll=1)
        def xrow(r):
            base = LPAD + r * 64
            for p in range(4):
                o = base + p * 16
                cv = lab[pl.ds(o, 16)]
                lv = lab[pl.ds(o - 1, 16)]
                rv = lab[pl.ds(o + 1, 16)]
                if p == 0:
                    lv = jnp.where(io == 0, cv, lv)
                if p == 3:
                    rv = jnp.where(io == 15, cv, rv)
                oo = r * 64 + p * 16
                mnx[pl.ds(oo, 16)] = jnp.minimum(jnp.minimum(lv, cv), rv)
                mxx[pl.ds(oo, 16)] = jnp.maximum(jnp.maximum(lv, cv), rv)

        @plsc.parallel_loop(0, 64, unroll=2)
        def yrow(r):
            rm = jnp.maximum(r - 1, 0) * 64
            rc = r * 64
            rp = jnp.minimum(r + 1, 63) * 64
            for p in range(4):
                q = p * 16
                mnr[pl.ds(rc + q, 16)] = jnp.minimum(
                    jnp.minimum(mnx[pl.ds(rm + q, 16)], mnx[pl.ds(rc + q, 16)]),
                    mnx[pl.ds(rp + q, 16)])
                mxr[pl.ds(rc + q, 16)] = jnp.maximum(
                    jnp.maximum(mxx[pl.ds(rm + q, 16)], mxx[pl.ds(rc + q, 16)]),
                    mxx[pl.ds(rp + q, 16)])

    load_lab(z0 - 1, 0)
    xy_pass(0)
    load_lab(z0, 1)
    xy_pass(1)

    for k in range(ZPW):
        z = z0 + k
        sl_cur = (k + 1) % 3
        sl_next = (k + 2) % 3

        load_lab(z + 1, sl_next)
        xy_pass(sl_next)

        mnp_, mnc_, mnn_ = mns[k % 3], mns[sl_cur], mns[sl_next]
        mxp_, mxc_, mxn_ = mxs[k % 3], mxs[sl_cur], mxs[sl_next]
        labc_ = labs[sl_cur]

        for cp in cps:
            cp.wait()

        _mn = (mnp_, mnc_, mnn_)
        _mx = (mxp_, mxc_, mxn_)
        _lab = labc_

        @plsc.parallel_loop(0, PLANE // 16, unroll=2)
        def achunk(i):
            po = i * 16
            mnv = jnp.minimum(
                jnp.minimum(_mn[0][pl.ds(po, 16)],
                            _mn[1][pl.ds(po, 16)]),
                _mn[2][pl.ds(po, 16)])
            mxv = jnp.maximum(
                jnp.maximum(_mx[0][pl.ds(po, 16)],
                            _mx[1][pl.ds(po, 16)]),
                _mx[2][pl.ds(po, 16)])
            wv = jnp.where(mxv != mnv, W_EDGE, 1.0)
            wbuf[pl.ds(po, 16)] = wv
            labv = _lab[pl.ds(LPAD + po, 16)]
            base = labv * 16 + io
            plsc.addupdate_scatter(rep, [base + 16 * 256], wv)
            plsc.addupdate_scatter(rep, [base + 17 * 256], onef)
            for e in range(E):
                v = emb[pl.ds(e * PLANE + po, 16)]
                plsc.addupdate_scatter(rep, [base + e * 256], wv * v)
        if k + 1 < ZPW:
            cps = fire_full(k + 1)

        pltpu.sync_copy(wbuf, w_out.at[pl.ds((b * Z + z) * PLANE, PLANE)])

    cps = fire(0)

    def fold(g, _):
        acc = zf
        for l in range(16):
            acc = acc + plsc.load_gather(rep, [g * 256 + io * 16 + l])
        part_v[pl.ds(g * 16, 16)] = acc
        return 0
    lax.fori_loop(0, 18, fold, 0)

    pltpu.sync_copy(part_v, shared.at[pl.ds(s * PART, PART)])
    plsc.subcore_barrier()
    pltpu.sync_copy(shared, rep.at[pl.ds(0, NS * PART)])

    def redrow(g):
        def red(i, acc, _g=g):
            return acc + rep[pl.ds(i * PART + _g * 16, 16)]
        return lax.fori_loop(0, NS, red, zf)

    wsum_vec = redrow(16) + 1e-8
    cnt_vec = redrow(17)
    for e in range(E):
        centers_v[pl.ds(e * 16, 16)] = redrow(e) / wsum_vec

    def crow(e, _):
        row = centers_v[pl.ds(e * 16, 16)]
        for l in range(16):
            plsc.store_scatter(crep, [e * 256 + io * 16 + l], row)
        return 0
    lax.fori_loop(0, E, crow, 0)

    @pl.when(s == 0)
    def _():
        cent_t[pl.ds(0, 16)] = zf
        for uid in range(1, 9):
            cent_t[pl.ds(uid * 16, 16)] = plsc.load_gather(
                centers_v, [io * 16 + uid])

    for r in range(16):
        rep[pl.ds(r * 16, 16)] = zf

    for t in range(NSEG):
        k, h = t // 2, t % 2
        z = z0 + k
        if h == 0:
            pltpu.sync_copy(label.at[pl.ds((b * Z + z) * PLANE, PLANE)],
                            labs[0].at[pl.ds(0, PLANE)])
            pltpu.sync_copy(w_out.at[pl.ds((b * Z + z) * PLANE, PLANE)],
                            wbuf)
        for cp in cps:
            cp.wait()
        if t + 1 < NSEG:
            cps = fire(t + 1)
        eb = (t % 2) * (E * HPLANE)
        hb = h * HPLANE

        _hb, _eb = hb, eb

        @plsc.parallel_loop(0, HPLANE // 16, unroll=2)
        def pchunk(i):
            o = i * 16
            po = _hb + o
            labv = labs[0][pl.ds(po, 16)]
            wv = wbuf[pl.ds(po, 16)]
            base = labv * 16 + io
            sq = []
            for e in range(E):
                v = emb[pl.ds(_eb + e * HPLANE + o, 16)]
                cs = plsc.load_gather(crep, [base + e * 256])
                d = v - cs
                sq.append(d * d)
            while len(sq) > 1:
                sq = [a + bq for a, bq in zip(sq[::2], sq[1::2])]
            dist = _nsqrt(sq[0])
            t_ = jnp.maximum(dist - D_V, 0.0)
            ww = jnp.where(labv == 0, 0.0, wv)
            plsc.addupdate_scatter(rep, [base], t_ * t_ * ww)

    acc = zf
    for l in range(16):
        acc = acc + plsc.load_gather(rep, [io * 16 + l])
    part_v[pl.ds(0, 16)] = acc
    pltpu.sync_copy(part_v.at[pl.ds(0, 16)], shared.at[pl.ds(s * PART, 16)])
    plsc.subcore_barrier()

    @pl.when(s == 0)
    def _():
        pltpu.sync_copy(shared, rep.at[pl.ds(0, NS * PART)])
        def redp(i, a):
            return a + rep[pl.ds(i * PART, 16)]
        pull_vec = lax.fori_loop(0, NS, redp, zf)
        uidmask = (io >= 1) & (io <= 8)
        q = jnp.where(uidmask, pull_vec / cnt_vec, 0.0)
        loss = jnp.sum(q) * W_PULL
        push = jnp.float32(0.0)
        for i in range(1, 8):
            def pj(j, a, _i=i):
                d = (cent_t[pl.ds(_i * 16, 16)]
                     - cent_t[pl.ds(j * 16, 16)])
                dd = jnp.max(_nsqrt(jnp.full((16,), jnp.sum(d * d))))
                pp = jnp.maximum(2.0 * D_D - dd, 0.0)
                return a + pp * pp
            push = lax.fori_loop(i + 1, 9, pj, push)
        loss = loss + push * (W_PUSH / 28.0)
        norm = jnp.float32(0.0)
        for i in range(1, 9):
            cv = cent_t[pl.ds(i * 16, 16)]
            norm = norm + jnp.max(_nsqrt(jnp.full((16,), jnp.sum(cv * cv))))
        loss = loss + norm * (W_NORM * 0.125)
        part_v[pl.ds(0, 16)] = jnp.full((16,), loss)
        pltpu.sync_copy(part_v.at[pl.ds(0, 16)],
                        loss_out.at[pl.ds(b * 16, 16)])


def kernel(embed, label):
    embed_f = embed.reshape(-1)
    label_f = label.reshape(-1)
    loss, _w = _kern_ab(embed_f, label_f)
    return (loss[0] + loss[16]) * jnp.float32(1.0 / B)
